# baseline (speedup 1.0000x reference)
import functools

import jax
import jax.numpy as jnp
from jax import lax
from jax.experimental import pallas as pl
from jax.experimental.pallas import tpu as pltpu
from jax.experimental.pallas import tpu_sc as plsc

N = 320000
D = 128
C = 10000
B = 64
NBLK = N // B
NC = 2
NS = 16
NW = NC * NS
FULL = NBLK // NW
EXTRA = NBLK % NW
TILE_SHARE = 624
TAIL_ROWS = C - NS * TILE_SHARE


def _count_chunks(ib, cnt):
  for k in range(B // 16):
    ids = ib[pl.ds(k * 16, 16)]
    run, last = plsc.scan_count(ids)
    plsc.addupdate_scatter(cnt, [ids], run.astype(jnp.float32), mask=last)


def _sc_body(x_hbm, sl_hbm, z128_hbm, psums, pcnts,
             sums_s, xb0, xb1, xb2, ib0, ib1, ib2, cnt,
             sx0, sx1, sx2, si0, si1, si2, ss0, ss1, ss2):
  cid = lax.axis_index("c")
  sid = lax.axis_index("s")
  wid = sid * NC + cid
  nvalid = jnp.where(wid < EXTRA, FULL + 1, FULL)
  xbs = (xb0, xb1, xb2)
  ibs = (ib0, ib1, ib2)
  sxs = (sx0, sx1, sx2)
  sis = (si0, si1, si2)
  sss = (ss0, ss1, ss2)

  def start_in(blk, p):
    off = pl.multiple_of(blk * B, B)
    pltpu.async_copy(x_hbm.at[pl.ds(off, B)], xbs[p], sxs[p])
    pltpu.async_copy(sl_hbm.at[pl.ds(off, B)], ibs[p], sis[p])

  def wait_in(p):
    pltpu.make_async_copy(x_hbm.at[pl.ds(0, B)], xbs[p], sxs[p]).wait()
    pltpu.make_async_copy(sl_hbm.at[pl.ds(0, B)], ibs[p], sis[p]).wait()

  def wait_scat(p):
    pltpu.make_async_copy(xbs[p], sums_s.at[ibs[p]], sss[p]).wait()

  start_in(wid, 0)
  start_in(wid + NW, 1)

  base = pl.multiple_of(sid * TILE_SHARE, 8)
  for k in range(39):
    off = pl.multiple_of(base + k * 16, 8)
    pltpu.sync_copy(z128_hbm, sums_s.at[pl.ds(off, 16)])

  @pl.when(sid == NS - 1)
  def _():
    pltpu.sync_copy(z128_hbm, sums_s.at[pl.ds(NS * TILE_SHARE, TAIL_ROWS)])

  def zero_cnt(i, _):
    cnt[pl.ds(i * 16, 16)] = jnp.zeros((16,), jnp.float32)
    return 0
  lax.fori_loop(0, C // 16, zero_cnt, 0)

  plsc.subcore_barrier()

  def step(t, _):
    for par in range(3):
      j = 3 * t + par
      wait_in(par)
      pltpu.async_copy(xbs[par], sums_s.at[ibs[par]], sss[par], add=True)
      _count_chunks(ibs[par], cnt)
      pb = (par + 2) % 3
      if par == 0:
        @pl.when(t > 0)
        def _():
          wait_scat(pb)
      else:
        wait_scat(pb)

      nxt = j + 2

      @pl.when(nxt < nvalid)
      def _():
        start_in(nxt * NW + wid, pb)
    return 0
  lax.fori_loop(0, FULL // 3, step, 0)

  @pl.when(wid < EXTRA)
  def _():
    wait_in(0)
    pltpu.async_copy(xbs[0], sums_s.at[ibs[0]], sss[0], add=True)
    _count_chunks(ibs[0], cnt)
    wait_scat(0)

  wait_scat(2)

  plsc.subcore_barrier()

  pltpu.sync_copy(sums_s.at[pl.ds(base, TILE_SHARE)],
                  psums.at[cid, pl.ds(base, TILE_SHARE)])

  @pl.when(sid == NS - 1)
  def _():
    pltpu.sync_copy(sums_s.at[pl.ds(NS * TILE_SHARE, TAIL_ROWS)],
                    psums.at[cid, pl.ds(NS * TILE_SHARE, TAIL_ROWS)])

  cof = pl.multiple_of(wid * C, 8)
  pltpu.sync_copy(cnt, pcnts.at[pl.ds(cof, C)])


_sc_stage = functools.partial(
    pl.kernel,
    out_type=(jax.ShapeDtypeStruct((NC, C, D), jnp.float32),
              jax.ShapeDtypeStruct((NW * C,), jnp.float32)),
    mesh=plsc.VectorSubcoreMesh(core_axis_name="c", subcore_axis_name="s"),
    compiler_params=pltpu.CompilerParams(needs_layout_passes=False),
    scratch_types=(
        pltpu.VMEM_SHARED((C, D), jnp.float32),
        pltpu.VMEM((B, D), jnp.float32),
        pltpu.VMEM((B, D), jnp.float32),
        pltpu.VMEM((B, D), jnp.float32),
        pltpu.VMEM((B,), jnp.int32),
        pltpu.VMEM((B,), jnp.int32),
        pltpu.VMEM((B,), jnp.int32),
        pltpu.VMEM((C,), jnp.float32),
        pltpu.SemaphoreType.DMA,
        pltpu.SemaphoreType.DMA,
        pltpu.SemaphoreType.DMA,
        pltpu.SemaphoreType.DMA,
        pltpu.SemaphoreType.DMA,
        pltpu.SemaphoreType.DMA,
        pltpu.SemaphoreType.DMA,
        pltpu.SemaphoreType.DMA,
        pltpu.SemaphoreType.DMA,
    ),
)(_sc_body)


def _div_body(ps_ref, pc_ref, out_ref):
  s = ps_ref[0] + ps_ref[1]
  n = jnp.sum(pc_ref[...], axis=0)
  out_ref[...] = s / n[:, None]


def kernel(x, sl):
  sl32 = sl.astype(jnp.int32)
  z128 = jnp.zeros((16, D), jnp.float32)
  psums, pcnts = _sc_stage(x, sl32, z128)
  out = pl.pallas_call(
      _div_body,
      out_shape=jax.ShapeDtypeStruct((C, D), jnp.float32),
  )(psums, pcnts.reshape(NW, C))
  return out


# speedup vs baseline: 1.3165x; 1.3165x over previous
import functools

import jax
import jax.numpy as jnp
from jax import lax
from jax.experimental import pallas as pl
from jax.experimental.pallas import tpu as pltpu
from jax.experimental.pallas import tpu_sc as plsc

N = 320000
D = 128
C = 10000
B = 64
NBLK = N // B
NC = 2
NS = 16
NW = NC * NS
FULL = NBLK // NW
EXTRA = NBLK % NW
TILE_SHARE = 624
TAIL_ROWS = C - NS * TILE_SHARE


def _count_chunks(ib, cnt):
  for k in range(B // 16):
    ids = ib[pl.ds(k * 16, 16)]
    run, last = plsc.scan_count(ids)
    plsc.addupdate_scatter(cnt, [ids], run.astype(jnp.float32), mask=last)


def _sc_body(x_hbm, sl_hbm, z64_hbm, psums, pcnts,
             sums_s, xb0, xb1, xb2, ib0, ib1, ib2, cnt,
             sx0, sx1, sx2, si0, si1, si2, ss0, ss1, ss2):
  cid = lax.axis_index("c")
  sid = lax.axis_index("s")
  wid = sid * NC + cid
  nvalid = jnp.where(wid < EXTRA, FULL + 1, FULL)
  xbs = (xb0, xb1, xb2)
  ibs = (ib0, ib1, ib2)
  sxs = (sx0, sx1, sx2)
  sis = (si0, si1, si2)
  sss = (ss0, ss1, ss2)

  def start_in(blk, p):
    off = pl.multiple_of(blk * B, B)
    pltpu.async_copy(x_hbm.at[pl.ds(off, B)], xbs[p], sxs[p])
    pltpu.async_copy(sl_hbm.at[pl.ds(off, B)], ibs[p], sis[p])

  def wait_in(p):
    pltpu.make_async_copy(x_hbm.at[pl.ds(0, B)], xbs[p], sxs[p]).wait()
    pltpu.make_async_copy(sl_hbm.at[pl.ds(0, B)], ibs[p], sis[p]).wait()

  def wait_scat(p):
    pltpu.make_async_copy(xbs[p], sums_s.at[ibs[p]], sss[p]).wait()

  start_in(wid, 0)
  start_in(wid + NW, 1)

  pltpu.sync_copy(z64_hbm, xb2)
  base = pl.multiple_of(sid * TILE_SHARE, 8)
  for k in range(10):
    off = pl.multiple_of(base + k * B, 8)
    pltpu.sync_copy(xb2, sums_s.at[pl.ds(off, B)])

  def zero_cnt(i, _):
    cnt[pl.ds(i * 16, 16)] = jnp.zeros((16,), jnp.float32)
    return 0
  lax.fori_loop(0, C // 16, zero_cnt, 0)

  plsc.subcore_barrier()

  def step(t, _):
    for par in range(3):
      j = 3 * t + par
      wait_in(par)
      pltpu.async_copy(xbs[par], sums_s.at[ibs[par]], sss[par], add=True)
      _count_chunks(ibs[par], cnt)
      pb = (par + 2) % 3
      if par == 0:
        @pl.when(t > 0)
        def _():
          wait_scat(pb)
      else:
        wait_scat(pb)

      nxt = j + 2

      @pl.when(nxt < nvalid)
      def _():
        start_in(nxt * NW + wid, pb)
    return 0
  lax.fori_loop(0, FULL // 3, step, 0)

  @pl.when(wid < EXTRA)
  def _():
    wait_in(0)
    pltpu.async_copy(xbs[0], sums_s.at[ibs[0]], sss[0], add=True)
    _count_chunks(ibs[0], cnt)
    wait_scat(0)

  wait_scat(2)

  plsc.subcore_barrier()

  pltpu.sync_copy(sums_s.at[pl.ds(base, TILE_SHARE)],
                  psums.at[cid, pl.ds(base, TILE_SHARE)])

  @pl.when(sid == NS - 1)
  def _():
    pltpu.sync_copy(sums_s.at[pl.ds(NS * TILE_SHARE, TAIL_ROWS)],
                    psums.at[cid, pl.ds(NS * TILE_SHARE, TAIL_ROWS)])

  cof = pl.multiple_of(wid * C, 8)
  pltpu.sync_copy(cnt, pcnts.at[pl.ds(cof, C)])


_sc_stage = functools.partial(
    pl.kernel,
    out_type=(jax.ShapeDtypeStruct((NC, C, D), jnp.float32),
              jax.ShapeDtypeStruct((NW * C,), jnp.float32)),
    mesh=plsc.VectorSubcoreMesh(core_axis_name="c", subcore_axis_name="s"),
    compiler_params=pltpu.CompilerParams(needs_layout_passes=False),
    scratch_types=(
        pltpu.VMEM_SHARED((C, D), jnp.float32),
        pltpu.VMEM((B, D), jnp.float32),
        pltpu.VMEM((B, D), jnp.float32),
        pltpu.VMEM((B, D), jnp.float32),
        pltpu.VMEM((B,), jnp.int32),
        pltpu.VMEM((B,), jnp.int32),
        pltpu.VMEM((B,), jnp.int32),
        pltpu.VMEM((C,), jnp.float32),
        pltpu.SemaphoreType.DMA,
        pltpu.SemaphoreType.DMA,
        pltpu.SemaphoreType.DMA,
        pltpu.SemaphoreType.DMA,
        pltpu.SemaphoreType.DMA,
        pltpu.SemaphoreType.DMA,
        pltpu.SemaphoreType.DMA,
        pltpu.SemaphoreType.DMA,
        pltpu.SemaphoreType.DMA,
    ),
)(_sc_body)


def _div_body(ps_ref, pc_ref, out_ref):
  s = ps_ref[0] + ps_ref[1]
  n = jnp.sum(pc_ref[...], axis=0)
  out_ref[...] = s / n[:, None]


def kernel(x, sl):
  sl32 = sl.astype(jnp.int32)
  z64 = jnp.zeros((B, D), jnp.float32)
  psums, pcnts = _sc_stage(x, sl32, z64)
  out = pl.pallas_call(
      _div_body,
      out_shape=jax.ShapeDtypeStruct((C, D), jnp.float32),
  )(psums, pcnts.reshape(NW, C))
  return out


# speedup vs baseline: 1.3181x; 1.0012x over previous
import functools

import jax
import jax.numpy as jnp
from jax import lax
from jax.experimental import pallas as pl
from jax.experimental.pallas import tpu as pltpu
from jax.experimental.pallas import tpu_sc as plsc

N = 320000
D = 128
C = 10000
B = 80
NBLK = N // B
NC = 2
NS = 16
NW = NC * NS
FULL = NBLK // NW
EXTRA = NBLK % NW
TILE_SHARE = 624
TAIL_ROWS = C - NS * TILE_SHARE


def _count_chunks(ib, cnt):
  for k in range(B // 16):
    ids = ib[pl.ds(k * 16, 16)]
    run, last = plsc.scan_count(ids)
    plsc.addupdate_scatter(cnt, [ids], run.astype(jnp.float32), mask=last)


def _sc_body(x_hbm, sl_hbm, z64_hbm, psums, pcnts,
             sums_s, xb0, xb1, xb2, ib0, ib1, ib2, cnt,
             sx0, sx1, sx2, si0, si1, si2, ss0, ss1, ss2):
  cid = lax.axis_index("c")
  sid = lax.axis_index("s")
  wid = sid * NC + cid
  nvalid = jnp.where(wid < EXTRA, FULL + 1, FULL)
  xbs = (xb0, xb1, xb2)
  ibs = (ib0, ib1, ib2)
  sxs = (sx0, sx1, sx2)
  sis = (si0, si1, si2)
  sss = (ss0, ss1, ss2)

  def start_in(blk, p):
    off = pl.multiple_of(blk * B, B)
    pltpu.async_copy(x_hbm.at[pl.ds(off, B)], xbs[p], sxs[p])
    pltpu.async_copy(sl_hbm.at[pl.ds(off, B)], ibs[p], sis[p])

  def wait_in(p):
    pltpu.make_async_copy(x_hbm.at[pl.ds(0, B)], xbs[p], sxs[p]).wait()
    pltpu.make_async_copy(sl_hbm.at[pl.ds(0, B)], ibs[p], sis[p]).wait()

  def wait_scat(p):
    pltpu.make_async_copy(xbs[p], sums_s.at[ibs[p]], sss[p]).wait()

  start_in(wid, 0)
  start_in(wid + NW, 1)

  pltpu.sync_copy(z64_hbm, xb2)
  base = pl.multiple_of(sid * TILE_SHARE, 8)
  for k in range(640 // B):
    off = pl.multiple_of(base + k * B, 8)
    pltpu.sync_copy(xb2, sums_s.at[pl.ds(off, B)])

  def zero_cnt(i, _):
    cnt[pl.ds(i * 16, 16)] = jnp.zeros((16,), jnp.float32)
    return 0
  lax.fori_loop(0, C // 16, zero_cnt, 0)

  plsc.subcore_barrier()

  def step(t, _):
    for par in range(3):
      j = 3 * t + par
      wait_in(par)
      pltpu.async_copy(xbs[par], sums_s.at[ibs[par]], sss[par], add=True)
      pb = (par + 2) % 3
      if par == 0:
        @pl.when(t > 0)
        def _():
          wait_scat(pb)
      else:
        wait_scat(pb)

      nxt = j + 2

      @pl.when(nxt < nvalid)
      def _():
        start_in(nxt * NW + wid, pb)

      _count_chunks(ibs[par], cnt)
    return 0
  lax.fori_loop(0, FULL // 3, step, 0)

  for par in (0, 1):
    j = (FULL // 3) * 3 + par
    wait_in(par)
    pltpu.async_copy(xbs[par], sums_s.at[ibs[par]], sss[par], add=True)
    _count_chunks(ibs[par], cnt)

  wait_scat(2)
  wait_scat(0)
  wait_scat(1)

  plsc.subcore_barrier()

  pltpu.sync_copy(sums_s.at[pl.ds(base, TILE_SHARE)],
                  psums.at[cid, pl.ds(base, TILE_SHARE)])

  @pl.when(sid == NS - 1)
  def _():
    pltpu.sync_copy(sums_s.at[pl.ds(NS * TILE_SHARE, TAIL_ROWS)],
                    psums.at[cid, pl.ds(NS * TILE_SHARE, TAIL_ROWS)])

  cof = pl.multiple_of(wid * C, 8)
  pltpu.sync_copy(cnt, pcnts.at[pl.ds(cof, C)])


_sc_stage = functools.partial(
    pl.kernel,
    out_type=(jax.ShapeDtypeStruct((NC, C, D), jnp.float32),
              jax.ShapeDtypeStruct((NW * C,), jnp.float32)),
    mesh=plsc.VectorSubcoreMesh(core_axis_name="c", subcore_axis_name="s"),
    compiler_params=pltpu.CompilerParams(needs_layout_passes=False),
    scratch_types=(
        pltpu.VMEM_SHARED((C, D), jnp.float32),
        pltpu.VMEM((B, D), jnp.float32),
        pltpu.VMEM((B, D), jnp.float32),
        pltpu.VMEM((B, D), jnp.float32),
        pltpu.VMEM((B,), jnp.int32),
        pltpu.VMEM((B,), jnp.int32),
        pltpu.VMEM((B,), jnp.int32),
        pltpu.VMEM((C,), jnp.float32),
        pltpu.SemaphoreType.DMA,
        pltpu.SemaphoreType.DMA,
        pltpu.SemaphoreType.DMA,
        pltpu.SemaphoreType.DMA,
        pltpu.SemaphoreType.DMA,
        pltpu.SemaphoreType.DMA,
        pltpu.SemaphoreType.DMA,
        pltpu.SemaphoreType.DMA,
        pltpu.SemaphoreType.DMA,
    ),
)(_sc_body)


def _div_body(ps_ref, pc_ref, out_ref):
  s = ps_ref[0] + ps_ref[1]
  n = jnp.sum(pc_ref[...], axis=0)
  out_ref[...] = s / n[:, None]


def kernel(x, sl):
  sl32 = sl.astype(jnp.int32)
  z64 = jnp.zeros((B, D), jnp.float32)
  psums, pcnts = _sc_stage(x, sl32, z64)
  out = pl.pallas_call(
      _div_body,
      out_shape=jax.ShapeDtypeStruct((C, D), jnp.float32),
  )(psums, pcnts.reshape(NW, C))
  return out
